# baseline (speedup 1.0000x reference)
import functools

import jax
import jax.numpy as jnp
from jax import lax
from jax.experimental import pallas as pl
from jax.experimental.pallas import tpu as pltpu
from jax.experimental.pallas import tpu_sc as plsc

NC = 2
NS = 16
NW = NC * NS
L = 16
D = 128
B = 128

BITREV = [0, 8, 4, 12, 2, 10, 6, 14, 1, 9, 5, 13, 3, 11, 7, 15]


def _make_sc_call(ept, n_chunks):
    mesh = plsc.VectorSubcoreMesh(core_axis_name="c", subcore_axis_name="s")

    @functools.partial(
        pl.kernel,
        mesh=mesh,
        compiler_params=pltpu.CompilerParams(needs_layout_passes=False,
                                             disable_bounds_checks=True,
                                             use_tc_tiling_on_sc=False),
        out_type=jax.ShapeDtypeStruct((NW * ept,), jnp.float32),
        scratch_types=[
            pltpu.VMEM((ept,), jnp.int32),
            pltpu.VMEM((ept,), jnp.int32),
            pltpu.VMEM((B, D // 2), jnp.int32),
            pltpu.VMEM((B, D // 2), jnp.int32),
            pltpu.VMEM((B, D // 2), jnp.int32),
            pltpu.VMEM((B, D // 2), jnp.int32),
            pltpu.VMEM((ept,), jnp.float32),
            pltpu.SemaphoreType.DMA,
            pltpu.SemaphoreType.DMA,
            pltpu.SemaphoreType.DMA,
            pltpu.SemaphoreType.DMA,
        ],
    )
    def sc_kernel(team_hbm, expert_hbm, tidx_hbm, eidx_hbm, out_hbm,
                  tidx_v, eidx_v, rows_t0, rows_e0, rows_t1, rows_e1,
                  out_v, sem0, sem1, sem2, sem3):
        wid = lax.axis_index("s") * NC + lax.axis_index("c")
        base = wid * ept
        pltpu.sync_copy(tidx_hbm.at[pl.ds(base, ept)], tidx_v)
        pltpu.sync_copy(eidx_hbm.at[pl.ds(base, ept)], eidx_v)

        lanes = lax.iota(jnp.int32, L)
        perm_idx = {}
        half_mask = {}
        g = L
        while g > 1:
            perm_idx[g] = (lanes & ~(g - 1)) | ((lanes + g // 2) & (g - 1))
            half_mask[g] = (lanes & (g // 2)) == 0
            g //= 2

        def lane_perm(v, idx):
            return jnp.take_along_axis(v, idx, axis=0,
                                       mode="promise_in_bounds")

        def compute_chunk(g, rows_t, rows_e):
            off = g * B

            def group_body(grp, _):
                regs = []
                for j in BITREV:
                    e = grp * L + j
                    prods = []
                    for k in range(D // (2 * L)):
                        tk = plsc.bitcast(rows_t[e, pl.ds(k * L, L)],
                                          jnp.bfloat16)
                        ek = plsc.bitcast(rows_e[e, pl.ds(k * L, L)],
                                          jnp.bfloat16)
                        ta, tb = plsc.unpack(
                            tk, format=plsc.PackFormat.INTERLEAVED)
                        ea, eb = plsc.unpack(
                            ek, format=plsc.PackFormat.INTERLEAVED)
                        prods.append(ta * ea)
                        prods.append(tb * eb)
                    while len(prods) > 1:
                        prods = [a + b for a, b in
                                 zip(prods[::2], prods[1::2])]
                    regs.append(prods[0])
                gg = L
                while len(regs) > 1:
                    nregs = []
                    for i in range(0, len(regs), 2):
                        ru = regs[i] + lane_perm(regs[i], perm_idx[gg])
                        rv = regs[i + 1] + lane_perm(regs[i + 1],
                                                     perm_idx[gg])
                        nregs.append(jnp.where(half_mask[gg], ru, rv))
                    regs = nregs
                    gg //= 2
                out_v[pl.ds(off + grp * L, L)] = regs[0]
                return 0

            lax.fori_loop(0, B // L, group_body, 0)

        bufs = ((rows_t0, rows_e0, sem0), (rows_t1, rows_e1, sem1))

        def start_chunk(off, rt, re, st, se):
            pltpu.async_copy(team_hbm.at[tidx_v.at[pl.ds(off, B)]], rt, st)
            pltpu.async_copy(expert_hbm.at[eidx_v.at[pl.ds(off, B)]], re, se)

        def wait_chunk(off, rt, re, st, se):
            pltpu.make_async_copy(team_hbm.at[tidx_v.at[pl.ds(off, B)]],
                                  rt, st).wait()
            pltpu.make_async_copy(expert_hbm.at[eidx_v.at[pl.ds(off, B)]],
                                  re, se).wait()

        start_chunk(0, rows_t0, rows_e0, sem0, sem1)

        def chunk_body(g, _):
            off = g * B
            noff = jnp.minimum(off + B, (n_chunks - 1) * B)
            rt0, re0, st0, se0 = rows_t0, rows_e0, sem0, sem1
            rt1, re1, st1, se1 = rows_t1, rows_e1, sem2, sem3
            is_even = g % 2 == 0

            @pl.when(is_even)
            def _():
                wait_chunk(off, rt0, re0, st0, se0)
                start_chunk(noff, rt1, re1, st1, se1)
                compute_chunk(g, rt0, re0)

            @pl.when(jnp.logical_not(is_even))
            def _():
                wait_chunk(off, rt1, re1, st1, se1)
                start_chunk(noff, rt0, re0, st0, se0)
                compute_chunk(g, rt1, re1)

            return 0

        lax.fori_loop(0, n_chunks, chunk_body, 0)
        pltpu.sync_copy(out_v, out_hbm.at[pl.ds(base, ept)])

    return sc_kernel


def kernel(x_expert, x_team, edge_label_index_team_experts):
    n_edges = edge_label_index_team_experts.shape[1]
    grain = NW * B
    n_pad = (n_edges + grain - 1) // grain * grain
    ept = n_pad // NW

    tidx = edge_label_index_team_experts[0]
    eidx = edge_label_index_team_experts[1]
    if n_pad != n_edges:
        pad = (0, n_pad - n_edges)
        tidx = jnp.pad(tidx, pad)
        eidx = jnp.pad(eidx, pad)

    def pack_bf16(x):
        n, d = x.shape
        xb = x.astype(jnp.bfloat16).reshape(n, d // 2, 2)
        return lax.bitcast_convert_type(xb, jnp.int32)

    out = _make_sc_call(ept, ept // B)(
        pack_bf16(x_team), pack_bf16(x_expert), tidx, eidx)
    return out[:n_edges]


# speedup vs baseline: 1.0104x; 1.0104x over previous
import functools

import jax
import jax.numpy as jnp
from jax import lax
from jax.experimental import pallas as pl
from jax.experimental.pallas import tpu as pltpu
from jax.experimental.pallas import tpu_sc as plsc

NC = 2
NS = 16
NW = NC * NS
L = 16
D = 128
B = 128

BITREV = [0, 8, 4, 12, 2, 10, 6, 14, 1, 9, 5, 13, 3, 11, 7, 15]


def _make_sc_call(ept, n_chunks):
    mesh = plsc.VectorSubcoreMesh(core_axis_name="c", subcore_axis_name="s")

    @functools.partial(
        pl.kernel,
        mesh=mesh,
        compiler_params=pltpu.CompilerParams(needs_layout_passes=False,
                                             disable_bounds_checks=True,
                                             use_tc_tiling_on_sc=False),
        out_type=jax.ShapeDtypeStruct((NW * ept,), jnp.float32),
        scratch_types=[
            pltpu.VMEM((ept,), jnp.int32),
            pltpu.VMEM((ept,), jnp.int32),
            pltpu.VMEM((B, D // 2), jnp.int32),
            pltpu.VMEM((B, D // 2), jnp.int32),
            pltpu.VMEM((B, D // 2), jnp.int32),
            pltpu.VMEM((B, D // 2), jnp.int32),
            pltpu.VMEM((ept,), jnp.float32),
            pltpu.SemaphoreType.DMA,
            pltpu.SemaphoreType.DMA,
            pltpu.SemaphoreType.DMA,
            pltpu.SemaphoreType.DMA,
        ],
    )
    def sc_kernel(team_hbm, expert_hbm, tidx_hbm, eidx_hbm, out_hbm,
                  tidx_v, eidx_v, rows_t0, rows_e0, rows_t1, rows_e1,
                  out_v, sem0, sem1, sem2, sem3):
        wid = lax.axis_index("s") * NC + lax.axis_index("c")
        base = wid * ept
        pltpu.sync_copy(tidx_hbm.at[pl.ds(base, ept)], tidx_v)
        pltpu.sync_copy(eidx_hbm.at[pl.ds(base, ept)], eidx_v)

        lanes = lax.iota(jnp.int32, L)
        perm_idx = {}
        half_mask = {}
        g = L
        while g > 1:
            perm_idx[g] = (lanes & ~(g - 1)) | ((lanes + g // 2) & (g - 1))
            half_mask[g] = (lanes & (g // 2)) == 0
            g //= 2

        def lane_perm(v, idx):
            return jnp.take_along_axis(v, idx, axis=0,
                                       mode="promise_in_bounds")

        def compute_chunk(g, rows_t, rows_e):
            off = g * B

            def group_body(grp, _):
                regs = []
                if True:
                    out_v[pl.ds(off + grp * L, L)] = jnp.zeros((L,),
                                                               jnp.float32)
                    return 0
                for j in BITREV:
                    e = grp * L + j
                    prods = []
                    for k in range(D // (2 * L)):
                        tk = plsc.bitcast(rows_t[e, pl.ds(k * L, L)],
                                          jnp.bfloat16)
                        ek = plsc.bitcast(rows_e[e, pl.ds(k * L, L)],
                                          jnp.bfloat16)
                        ta, tb = plsc.unpack(
                            tk, format=plsc.PackFormat.INTERLEAVED)
                        ea, eb = plsc.unpack(
                            ek, format=plsc.PackFormat.INTERLEAVED)
                        prods.append(ta * ea)
                        prods.append(tb * eb)
                    while len(prods) > 1:
                        prods = [a + b for a, b in
                                 zip(prods[::2], prods[1::2])]
                    regs.append(prods[0])
                gg = L
                while len(regs) > 1:
                    nregs = []
                    for i in range(0, len(regs), 2):
                        ru = regs[i] + lane_perm(regs[i], perm_idx[gg])
                        rv = regs[i + 1] + lane_perm(regs[i + 1],
                                                     perm_idx[gg])
                        nregs.append(jnp.where(half_mask[gg], ru, rv))
                    regs = nregs
                    gg //= 2
                out_v[pl.ds(off + grp * L, L)] = regs[0]
                return 0

            lax.fori_loop(0, B // L, group_body, 0)

        bufs = ((rows_t0, rows_e0, sem0), (rows_t1, rows_e1, sem1))

        def start_chunk(off, rt, re, st, se):
            pltpu.async_copy(team_hbm.at[tidx_v.at[pl.ds(off, B)]], rt, st)
            pltpu.async_copy(expert_hbm.at[eidx_v.at[pl.ds(off, B)]], re, se)

        def wait_chunk(off, rt, re, st, se):
            pltpu.make_async_copy(team_hbm.at[tidx_v.at[pl.ds(off, B)]],
                                  rt, st).wait()
            pltpu.make_async_copy(expert_hbm.at[eidx_v.at[pl.ds(off, B)]],
                                  re, se).wait()

        start_chunk(0, rows_t0, rows_e0, sem0, sem1)

        def chunk_body(g, _):
            off = g * B
            noff = jnp.minimum(off + B, (n_chunks - 1) * B)
            rt0, re0, st0, se0 = rows_t0, rows_e0, sem0, sem1
            rt1, re1, st1, se1 = rows_t1, rows_e1, sem2, sem3
            is_even = g % 2 == 0

            @pl.when(is_even)
            def _():
                wait_chunk(off, rt0, re0, st0, se0)
                start_chunk(noff, rt1, re1, st1, se1)
                compute_chunk(g, rt0, re0)

            @pl.when(jnp.logical_not(is_even))
            def _():
                wait_chunk(off, rt1, re1, st1, se1)
                start_chunk(noff, rt0, re0, st0, se0)
                compute_chunk(g, rt1, re1)

            return 0

        lax.fori_loop(0, n_chunks, chunk_body, 0)
        pltpu.sync_copy(out_v, out_hbm.at[pl.ds(base, ept)])

    return sc_kernel


def kernel(x_expert, x_team, edge_label_index_team_experts):
    n_edges = edge_label_index_team_experts.shape[1]
    grain = NW * B
    n_pad = (n_edges + grain - 1) // grain * grain
    ept = n_pad // NW

    tidx = edge_label_index_team_experts[0]
    eidx = edge_label_index_team_experts[1]
    if n_pad != n_edges:
        pad = (0, n_pad - n_edges)
        tidx = jnp.pad(tidx, pad)
        eidx = jnp.pad(eidx, pad)

    def pack_bf16(x):
        n, d = x.shape
        xb = x.astype(jnp.bfloat16).reshape(n, d // 2, 2)
        return lax.bitcast_convert_type(xb, jnp.int32)

    out = _make_sc_call(ept, ept // B)(
        pack_bf16(x_team), pack_bf16(x_expert), tidx, eidx)
    return out[:n_edges]


# speedup vs baseline: 1.6226x; 1.6059x over previous
import functools

import jax
import jax.numpy as jnp
from jax import lax
from jax.experimental import pallas as pl
from jax.experimental.pallas import tpu as pltpu
from jax.experimental.pallas import tpu_sc as plsc

NC = 2
NS = 16
NW = NC * NS
L = 16
D = 128
B = 128

BITREV = [0, 8, 4, 12, 2, 10, 6, 14, 1, 9, 5, 13, 3, 11, 7, 15]


def _make_sc_call(ept, n_chunks):
    mesh = plsc.VectorSubcoreMesh(core_axis_name="c", subcore_axis_name="s")

    @functools.partial(
        pl.kernel,
        mesh=mesh,
        compiler_params=pltpu.CompilerParams(needs_layout_passes=False,
                                             disable_bounds_checks=True,
                                             use_tc_tiling_on_sc=False),
        out_type=jax.ShapeDtypeStruct((NW * ept,), jnp.float32),
        scratch_types=[
            pltpu.VMEM((ept,), jnp.int32),
            pltpu.VMEM((ept,), jnp.int32),
            pltpu.VMEM((B, D), jnp.float32),
            pltpu.VMEM((B, D), jnp.float32),
            pltpu.VMEM((B, D), jnp.float32),
            pltpu.VMEM((B, D), jnp.float32),
            pltpu.VMEM((ept,), jnp.float32),
            pltpu.SemaphoreType.DMA,
            pltpu.SemaphoreType.DMA,
            pltpu.SemaphoreType.DMA,
            pltpu.SemaphoreType.DMA,
        ],
    )
    def sc_kernel(team_hbm, expert_hbm, tidx_hbm, eidx_hbm, out_hbm,
                  tidx_v, eidx_v, rows_t0, rows_e0, rows_t1, rows_e1,
                  out_v, sem0, sem1, sem2, sem3):
        wid = lax.axis_index("s") * NC + lax.axis_index("c")
        base = wid * ept
        pltpu.sync_copy(tidx_hbm.at[pl.ds(base, ept)], tidx_v)
        pltpu.sync_copy(eidx_hbm.at[pl.ds(base, ept)], eidx_v)

        lanes = lax.iota(jnp.int32, L)
        perm_idx = {}
        half_mask = {}
        g = L
        while g > 1:
            perm_idx[g] = (lanes & ~(g - 1)) | ((lanes + g // 2) & (g - 1))
            half_mask[g] = (lanes & (g // 2)) == 0
            g //= 2

        def lane_perm(v, idx):
            return jnp.take_along_axis(v, idx, axis=0,
                                       mode="promise_in_bounds")

        def compute_chunk(g, rows_t, rows_e):
            off = g * B

            def group_body(grp, _):
                regs = []
                for j in BITREV:
                    e = grp * L + j
                    prods = [rows_t[e, pl.ds(k * L, L)] *
                             rows_e[e, pl.ds(k * L, L)]
                             for k in range(D // L)]
                    while len(prods) > 1:
                        prods = [a + b for a, b in
                                 zip(prods[::2], prods[1::2])]
                    regs.append(prods[0])
                gg = L
                while len(regs) > 1:
                    nregs = []
                    for i in range(0, len(regs), 2):
                        ru = regs[i] + lane_perm(regs[i], perm_idx[gg])
                        rv = regs[i + 1] + lane_perm(regs[i + 1],
                                                     perm_idx[gg])
                        nregs.append(jnp.where(half_mask[gg], ru, rv))
                    regs = nregs
                    gg //= 2
                out_v[pl.ds(off + grp * L, L)] = regs[0]
                return 0

            lax.fori_loop(0, B // L, group_body, 0)

        bufs = ((rows_t0, rows_e0, sem0), (rows_t1, rows_e1, sem1))

        def start_chunk(off, rt, re, st, se):
            pltpu.async_copy(team_hbm.at[tidx_v.at[pl.ds(off, B)]], rt, st)
            pltpu.async_copy(expert_hbm.at[eidx_v.at[pl.ds(off, B)]], re, se)

        def wait_chunk(off, rt, re, st, se):
            pltpu.make_async_copy(team_hbm.at[tidx_v.at[pl.ds(off, B)]],
                                  rt, st).wait()
            pltpu.make_async_copy(expert_hbm.at[eidx_v.at[pl.ds(off, B)]],
                                  re, se).wait()

        start_chunk(0, rows_t0, rows_e0, sem0, sem1)

        def chunk_body(g, _):
            off = g * B
            noff = jnp.minimum(off + B, (n_chunks - 1) * B)
            rt0, re0, st0, se0 = rows_t0, rows_e0, sem0, sem1
            rt1, re1, st1, se1 = rows_t1, rows_e1, sem2, sem3
            is_even = g % 2 == 0

            @pl.when(is_even)
            def _():
                wait_chunk(off, rt0, re0, st0, se0)
                start_chunk(noff, rt1, re1, st1, se1)
                compute_chunk(g, rt0, re0)

            @pl.when(jnp.logical_not(is_even))
            def _():
                wait_chunk(off, rt1, re1, st1, se1)
                start_chunk(noff, rt0, re0, st0, se0)
                compute_chunk(g, rt1, re1)

            return 0

        lax.fori_loop(0, n_chunks, chunk_body, 0)
        pltpu.sync_copy(out_v, out_hbm.at[pl.ds(base, ept)])

    return sc_kernel


def kernel(x_expert, x_team, edge_label_index_team_experts):
    n_edges = edge_label_index_team_experts.shape[1]
    grain = NW * B
    n_pad = (n_edges + grain - 1) // grain * grain
    ept = n_pad // NW

    tidx = edge_label_index_team_experts[0]
    eidx = edge_label_index_team_experts[1]
    if n_pad != n_edges:
        pad = (0, n_pad - n_edges)
        tidx = jnp.pad(tidx, pad)
        eidx = jnp.pad(eidx, pad)

    out = _make_sc_call(ept, ept // B)(x_team, x_expert, tidx, eidx)
    return out[:n_edges]


# speedup vs baseline: 2.0604x; 1.2698x over previous
import functools

import jax
import jax.numpy as jnp
from jax import lax
from jax.experimental import pallas as pl
from jax.experimental.pallas import tpu as pltpu
from jax.experimental.pallas import tpu_sc as plsc

NC = 2
NS = 16
NW = NC * NS
L = 16
D = 128
B = 128

BITREV = [0, 8, 4, 12, 2, 10, 6, 14, 1, 9, 5, 13, 3, 11, 7, 15]


def _make_sc_call(ept, n_chunks):
    mesh = plsc.VectorSubcoreMesh(core_axis_name="c", subcore_axis_name="s")

    @functools.partial(
        pl.kernel,
        mesh=mesh,
        compiler_params=pltpu.CompilerParams(needs_layout_passes=False,
                                             disable_bounds_checks=True,
                                             use_tc_tiling_on_sc=False),
        out_type=jax.ShapeDtypeStruct((NW * ept,), jnp.float32),
        scratch_types=[
            pltpu.VMEM((ept,), jnp.int32),
            pltpu.VMEM((ept,), jnp.int32),
            pltpu.VMEM((B, D // 2), jnp.int32),
            pltpu.VMEM((B, D // 2), jnp.int32),
            pltpu.VMEM((B, D // 2), jnp.int32),
            pltpu.VMEM((B, D // 2), jnp.int32),
            pltpu.VMEM((ept,), jnp.float32),
            pltpu.SemaphoreType.DMA,
            pltpu.SemaphoreType.DMA,
            pltpu.SemaphoreType.DMA,
            pltpu.SemaphoreType.DMA,
        ],
    )
    def sc_kernel(team_hbm, expert_hbm, tidx_hbm, eidx_hbm, out_hbm,
                  tidx_v, eidx_v, rows_t0, rows_e0, rows_t1, rows_e1,
                  out_v, sem0, sem1, sem2, sem3):
        wid = lax.axis_index("s") * NC + lax.axis_index("c")
        base = wid * ept
        pltpu.sync_copy(tidx_hbm.at[pl.ds(base, ept)], tidx_v)
        pltpu.sync_copy(eidx_hbm.at[pl.ds(base, ept)], eidx_v)

        lanes = lax.iota(jnp.int32, L)
        perm_idx = {}
        half_mask = {}
        g = L
        while g > 1:
            perm_idx[g] = (lanes & ~(g - 1)) | ((lanes + g // 2) & (g - 1))
            half_mask[g] = (lanes & (g // 2)) == 0
            g //= 2

        def lane_perm(v, idx):
            return jnp.take_along_axis(v, idx, axis=0,
                                       mode="promise_in_bounds")

        def compute_chunk(g, rows_t, rows_e):
            off = g * B

            def group_body(grp, _):
                regs = []
                for j in BITREV:
                    e = grp * L + j
                    prods = []
                    for k in range(D // (2 * L)):
                        tk = plsc.bitcast(rows_t[e, pl.ds(k * L, L)],
                                          jnp.bfloat16)
                        ek = plsc.bitcast(rows_e[e, pl.ds(k * L, L)],
                                          jnp.bfloat16)
                        ta, tb = plsc.unpack(
                            tk, format=plsc.PackFormat.INTERLEAVED)
                        ea, eb = plsc.unpack(
                            ek, format=plsc.PackFormat.INTERLEAVED)
                        prods.append(ta * ea)
                        prods.append(tb * eb)
                    while len(prods) > 1:
                        prods = [a + b for a, b in
                                 zip(prods[::2], prods[1::2])]
                    regs.append(prods[0])
                gg = L
                while len(regs) > 1:
                    nregs = []
                    for i in range(0, len(regs), 2):
                        ru = regs[i] + lane_perm(regs[i], perm_idx[gg])
                        rv = regs[i + 1] + lane_perm(regs[i + 1],
                                                     perm_idx[gg])
                        nregs.append(jnp.where(half_mask[gg], ru, rv))
                    regs = nregs
                    gg //= 2
                out_v[pl.ds(off + grp * L, L)] = regs[0]
                return 0

            lax.fori_loop(0, B // L, group_body, 0)

        bufs = ((rows_t0, rows_e0, sem0), (rows_t1, rows_e1, sem1))

        def start_chunk(off, rt, re, st, se):
            pltpu.async_copy(team_hbm.at[tidx_v.at[pl.ds(off, B)]], rt, st)
            pltpu.async_copy(expert_hbm.at[eidx_v.at[pl.ds(off, B)]], re, se)

        def wait_chunk(off, rt, re, st, se):
            pltpu.make_async_copy(team_hbm.at[tidx_v.at[pl.ds(off, B)]],
                                  rt, st).wait()
            pltpu.make_async_copy(expert_hbm.at[eidx_v.at[pl.ds(off, B)]],
                                  re, se).wait()

        start_chunk(0, rows_t0, rows_e0, sem0, sem1)

        def chunk_body(g, _):
            off = g * B
            noff = jnp.minimum(off + B, (n_chunks - 1) * B)
            rt0, re0, st0, se0 = rows_t0, rows_e0, sem0, sem1
            rt1, re1, st1, se1 = rows_t1, rows_e1, sem2, sem3
            is_even = g % 2 == 0

            @pl.when(is_even)
            def _():
                wait_chunk(off, rt0, re0, st0, se0)
                start_chunk(noff, rt1, re1, st1, se1)
                compute_chunk(g, rt0, re0)

            @pl.when(jnp.logical_not(is_even))
            def _():
                wait_chunk(off, rt1, re1, st1, se1)
                start_chunk(noff, rt0, re0, st0, se0)
                compute_chunk(g, rt1, re1)

            return 0

        lax.fori_loop(0, n_chunks, chunk_body, 0)
        pltpu.sync_copy(out_v, out_hbm.at[pl.ds(base, ept)])

    return sc_kernel


def kernel(x_expert, x_team, edge_label_index_team_experts):
    n_edges = edge_label_index_team_experts.shape[1]
    grain = NW * B
    n_pad = (n_edges + grain - 1) // grain * grain
    ept = n_pad // NW

    tidx = edge_label_index_team_experts[0]
    eidx = edge_label_index_team_experts[1]
    if n_pad != n_edges:
        pad = (0, n_pad - n_edges)
        tidx = jnp.pad(tidx, pad)
        eidx = jnp.pad(eidx, pad)

    def pack_bf16(x):
        xi = lax.bitcast_convert_type(x, jnp.int32)
        a = xi[:, :x.shape[1] // 2]
        b = xi[:, x.shape[1] // 2:]

        def rnd(v):
            return (v + 0x7FFF + ((v >> 16) & 1)) >> 16

        return (rnd(a) & 0xFFFF) | (rnd(b) << 16)

    out = _make_sc_call(ept, ept // B)(
        pack_bf16(x_team), pack_bf16(x_expert), tidx, eidx)
    return out[:n_edges]
